# baseline (speedup 1.0000x reference)
import dataclasses
import functools

import jax
import jax.numpy as jnp
from jax import lax
from jax.experimental import pallas as pl
from jax.experimental.pallas import tpu as pltpu
from jax.experimental.pallas import tpu_sc as plsc

N_NODES = 10000
N_EVENTS = 320000
D_FEAT = 128
TIME_DIM = 32
D_EDGE = 16
D_OUT = 2 * D_FEAT + TIME_DIM + D_EDGE

NUM_CORES = 2
NUM_SUBCORES = 16
NUM_LANES = 16
NUM_WORKERS = NUM_CORES * NUM_SUBCORES
EV_PER_WORKER = N_EVENTS // NUM_WORKERS
WIN = 400
N_WIN = EV_PER_WORKER // WIN

_MESH = plsc.VectorSubcoreMesh(core_axis_name="c", subcore_axis_name="s")


def _sc_cp(**kw):
    cp = pltpu.CompilerParams()
    fields = pltpu.CompilerParams.__dataclass_fields__
    return dataclasses.replace(cp, **{k: v for k, v in kw.items() if k in fields})


_UNTILED_CP = _sc_cp(needs_layout_passes=False, use_tc_tiling_on_sc=False)


def _sc_dt(last_update, timestamps, idx):
    @functools.partial(
        pl.kernel,
        out_type=jax.ShapeDtypeStruct((N_EVENTS,), jnp.float32),
        mesh=_MESH,
        scratch_types=[
            pltpu.VMEM((N_NODES,), jnp.float32),
            pltpu.VMEM((EV_PER_WORKER,), jnp.int32),
            pltpu.VMEM((EV_PER_WORKER,), jnp.float32),
            pltpu.VMEM((EV_PER_WORKER,), jnp.float32),
        ],
        compiler_params=_UNTILED_CP,
    )
    def k(lu_hbm, ts_hbm, idx_hbm, dt_hbm, lu_v, idx_v, ts_v, dt_v):
        wid = lax.axis_index("s") * NUM_CORES + lax.axis_index("c")
        base = wid * EV_PER_WORKER
        pltpu.sync_copy(lu_hbm, lu_v)
        pltpu.sync_copy(idx_hbm.at[pl.ds(base, EV_PER_WORKER)], idx_v)
        pltpu.sync_copy(ts_hbm.at[pl.ds(base, EV_PER_WORKER)], ts_v)

        @pl.loop(0, EV_PER_WORKER, step=NUM_LANES)
        def _(i):
            v = idx_v[pl.ds(i, NUM_LANES)]
            t16 = plsc.load_gather(lu_v, [v])
            dt_v[pl.ds(i, NUM_LANES)] = ts_v[pl.ds(i, NUM_LANES)] - t16

        pltpu.sync_copy(dt_v, dt_hbm.at[pl.ds(base, EV_PER_WORKER)])

    return k(last_update, timestamps, idx)


def _sc_feats(events_features, evt):
    @functools.partial(
        pl.kernel,
        out_type=jax.ShapeDtypeStruct((N_EVENTS, D_EDGE), jnp.float32),
        mesh=_MESH,
        scratch_types=[
            pltpu.VMEM((WIN,), jnp.int32),
            pltpu.VMEM((WIN, D_EDGE), jnp.float32),
            pltpu.SemaphoreType.DMA,
        ],
        compiler_params=_UNTILED_CP,
    )
    def k(feat_hbm, evt_hbm, ff_hbm, ei_v, frows_v, sem):
        wid = lax.axis_index("s") * NUM_CORES + lax.axis_index("c")
        base = wid * EV_PER_WORKER

        @pl.loop(0, N_WIN)
        def _(win):
            wbase = base + win * WIN
            pltpu.sync_copy(evt_hbm.at[pl.ds(wbase, WIN)], ei_v)
            pltpu.async_copy(feat_hbm.at[ei_v], frows_v, sem).wait()
            pltpu.sync_copy(frows_v, ff_hbm.at[pl.ds(wbase, WIN), :])

    return k(events_features, evt)


_BTC = 12800


def _delta_body(dt_ref, w_ref, b_ref, o_ref):
    i = pl.program_id(0)
    dtv = dt_ref[pl.ds(i * _BTC, _BTC)].reshape(1, _BTC)
    o_ref[...] = jnp.cos(w_ref[...] * dtv + b_ref[...])


def _tc_delta_t(dt, w_col, b_col):
    return pl.pallas_call(
        _delta_body,
        grid=(N_EVENTS // _BTC,),
        in_specs=[
            pl.BlockSpec((N_EVENTS,), lambda i: (0,)),
            pl.BlockSpec((TIME_DIM, 1), lambda i: (0, 0)),
            pl.BlockSpec((TIME_DIM, 1), lambda i: (0, 0)),
        ],
        out_specs=pl.BlockSpec((TIME_DIM, _BTC), lambda i: (0, i)),
        out_shape=jax.ShapeDtypeStruct((TIME_DIM, N_EVENTS), jnp.float32),
    )(dt, w_col, b_col)


def _sc_assemble(memory, src, dst):
    @functools.partial(
        pl.kernel,
        out_type=jax.ShapeDtypeStruct((N_EVENTS, D_OUT), jnp.float32),
        mesh=_MESH,
        scratch_types=[
            pltpu.VMEM((EV_PER_WORKER,), jnp.int32),
            pltpu.VMEM((EV_PER_WORKER,), jnp.int32),
            pltpu.VMEM((WIN, D_FEAT), jnp.float32),
            pltpu.VMEM((WIN, D_FEAT), jnp.float32),
            pltpu.SemaphoreType.DMA,
            pltpu.SemaphoreType.DMA,
        ],
    )
    def k(mem_hbm, src_hbm, dst_hbm, out_hbm,
          src_v, dst_v, buf_a, buf_b, sem_a, sem_b):
        wid = lax.axis_index("s") * NUM_CORES + lax.axis_index("c")
        base = wid * EV_PER_WORKER
        pltpu.sync_copy(src_hbm.at[pl.ds(base, EV_PER_WORKER)], src_v)
        pltpu.sync_copy(dst_hbm.at[pl.ds(base, EV_PER_WORKER)], dst_v)
        pltpu.async_copy(mem_hbm.at[src_v.at[pl.ds(0, WIN)]], buf_a, sem_a)

        @pl.loop(0, N_WIN)
        def _(win):
            wb = base + win * WIN
            pltpu.async_copy(
                mem_hbm.at[dst_v.at[pl.ds(win * WIN, WIN)]], buf_b, sem_b)
            pltpu.make_async_copy(
                mem_hbm.at[src_v.at[pl.ds(0, WIN)]], buf_a, sem_a).wait()
            pltpu.sync_copy(buf_a, out_hbm.at[pl.ds(wb, WIN), pl.ds(0, D_FEAT)])

            @pl.when(win + 1 < N_WIN)
            def _():
                pltpu.async_copy(
                    mem_hbm.at[src_v.at[pl.ds((win + 1) * WIN, WIN)]],
                    buf_a, sem_a)

            pltpu.make_async_copy(
                mem_hbm.at[dst_v.at[pl.ds(0, WIN)]], buf_b, sem_b).wait()
            pltpu.sync_copy(
                buf_b, out_hbm.at[pl.ds(wb, WIN), pl.ds(D_FEAT, D_FEAT)])

    return k(memory, src, dst)


def kernel(memory, last_update, events_features, time_w, time_b, timestamps,
           src_nodes, dst_nodes, event_indices, idx):
    dt = _sc_dt(last_update, timestamps, idx.astype(jnp.int32))
    ff = _sc_feats(events_features, event_indices.astype(jnp.int32))
    delta_t = _tc_delta_t(
        dt, time_w.reshape(TIME_DIM, 1), time_b.reshape(TIME_DIM, 1))
    out = _sc_assemble(
        memory, src_nodes.astype(jnp.int32), dst_nodes.astype(jnp.int32))
    out = lax.dynamic_update_slice(out, delta_t.T, (0, 2 * D_FEAT))
    out = lax.dynamic_update_slice(out, ff, (0, 2 * D_FEAT + TIME_DIM))
    return out


# speedup vs baseline: 2.1749x; 2.1749x over previous
import dataclasses
import functools

import jax
import jax.numpy as jnp
from jax import lax
from jax.experimental import pallas as pl
from jax.experimental.pallas import tpu as pltpu
from jax.experimental.pallas import tpu_sc as plsc

N_NODES = 10000
N_EVENTS = 320000
D_FEAT = 128
TIME_DIM = 32
D_EDGE = 16
D_OUT = 2 * D_FEAT + TIME_DIM + D_EDGE

NUM_CORES = 2
NUM_SUBCORES = 16
NUM_LANES = 16
NUM_WORKERS = NUM_CORES * NUM_SUBCORES
EV_PER_WORKER = N_EVENTS // NUM_WORKERS
WIN = 400
N_WIN = EV_PER_WORKER // WIN

_MESH = plsc.VectorSubcoreMesh(core_axis_name="c", subcore_axis_name="s")


def _sc_cp(**kw):
    cp = pltpu.CompilerParams()
    fields = pltpu.CompilerParams.__dataclass_fields__
    return dataclasses.replace(cp, **{k: v for k, v in kw.items() if k in fields})


_UNTILED_CP = _sc_cp(needs_layout_passes=False, use_tc_tiling_on_sc=False)


def _sc_dt(last_update, timestamps, idx):
    @functools.partial(
        pl.kernel,
        out_type=jax.ShapeDtypeStruct((N_EVENTS,), jnp.float32),
        mesh=_MESH,
        scratch_types=[
            pltpu.VMEM((N_NODES,), jnp.float32),
            pltpu.VMEM((EV_PER_WORKER,), jnp.int32),
            pltpu.VMEM((EV_PER_WORKER,), jnp.float32),
            pltpu.VMEM((EV_PER_WORKER,), jnp.float32),
        ],
        compiler_params=_UNTILED_CP,
    )
    def k(lu_hbm, ts_hbm, idx_hbm, dt_hbm, lu_v, idx_v, ts_v, dt_v):
        wid = lax.axis_index("s") * NUM_CORES + lax.axis_index("c")
        base = wid * EV_PER_WORKER
        pltpu.sync_copy(lu_hbm, lu_v)
        pltpu.sync_copy(idx_hbm.at[pl.ds(base, EV_PER_WORKER)], idx_v)
        pltpu.sync_copy(ts_hbm.at[pl.ds(base, EV_PER_WORKER)], ts_v)

        @pl.loop(0, EV_PER_WORKER, step=NUM_LANES)
        def _(i):
            v = idx_v[pl.ds(i, NUM_LANES)]
            t16 = plsc.load_gather(lu_v, [v])
            dt_v[pl.ds(i, NUM_LANES)] = ts_v[pl.ds(i, NUM_LANES)] - t16

        pltpu.sync_copy(dt_v, dt_hbm.at[pl.ds(base, EV_PER_WORKER)])

    return k(last_update, timestamps, idx)


def _sc_feats(events_features, evt):
    @functools.partial(
        pl.kernel,
        out_type=jax.ShapeDtypeStruct((N_EVENTS, D_EDGE), jnp.float32),
        mesh=_MESH,
        scratch_types=[
            pltpu.VMEM((WIN,), jnp.int32),
            pltpu.VMEM((WIN, D_EDGE), jnp.float32),
            pltpu.SemaphoreType.DMA,
        ],
        compiler_params=_UNTILED_CP,
    )
    def k(feat_hbm, evt_hbm, ff_hbm, ei_v, frows_v, sem):
        wid = lax.axis_index("s") * NUM_CORES + lax.axis_index("c")
        base = wid * EV_PER_WORKER

        @pl.loop(0, N_WIN)
        def _(win):
            wbase = base + win * WIN
            pltpu.sync_copy(evt_hbm.at[pl.ds(wbase, WIN)], ei_v)
            pltpu.async_copy(feat_hbm.at[ei_v], frows_v, sem).wait()
            pltpu.sync_copy(frows_v, ff_hbm.at[pl.ds(wbase, WIN), :])

    return k(events_features, evt)


_BTC = 12800


def _delta_body(dt_ref, w_ref, b_ref, o_ref):
    i = pl.program_id(0)
    dtv = dt_ref[pl.ds(i * _BTC, _BTC)].reshape(1, _BTC)
    o_ref[...] = jnp.cos(w_ref[...] * dtv + b_ref[...])


def _tc_delta_t(dt, w_col, b_col):
    return pl.pallas_call(
        _delta_body,
        grid=(N_EVENTS // _BTC,),
        in_specs=[
            pl.BlockSpec((N_EVENTS,), lambda i: (0,)),
            pl.BlockSpec((TIME_DIM, 1), lambda i: (0, 0)),
            pl.BlockSpec((TIME_DIM, 1), lambda i: (0, 0)),
        ],
        out_specs=pl.BlockSpec((TIME_DIM, _BTC), lambda i: (0, i)),
        out_shape=jax.ShapeDtypeStruct((TIME_DIM, N_EVENTS), jnp.float32),
    )(dt, w_col, b_col)


def _sc_assemble(memory, src, dst):
    @functools.partial(
        pl.kernel,
        out_type=jax.ShapeDtypeStruct((N_EVENTS, 2 * D_FEAT), jnp.float32),
        mesh=_MESH,
        scratch_types=[
            pltpu.VMEM((EV_PER_WORKER,), jnp.int32),
            pltpu.VMEM((EV_PER_WORKER,), jnp.int32),
            pltpu.VMEM((WIN, D_FEAT), jnp.float32),
            pltpu.VMEM((WIN, D_FEAT), jnp.float32),
            pltpu.SemaphoreType.DMA,
            pltpu.SemaphoreType.DMA,
        ],
    )
    def k(mem_hbm, src_hbm, dst_hbm, out_hbm,
          src_v, dst_v, buf_a, buf_b, sem_a, sem_b):
        wid = lax.axis_index("s") * NUM_CORES + lax.axis_index("c")
        base = wid * EV_PER_WORKER
        pltpu.sync_copy(src_hbm.at[pl.ds(base, EV_PER_WORKER)], src_v)
        pltpu.sync_copy(dst_hbm.at[pl.ds(base, EV_PER_WORKER)], dst_v)
        pltpu.async_copy(mem_hbm.at[src_v.at[pl.ds(0, WIN)]], buf_a, sem_a)

        @pl.loop(0, N_WIN)
        def _(win):
            wb = base + win * WIN
            pltpu.async_copy(
                mem_hbm.at[dst_v.at[pl.ds(win * WIN, WIN)]], buf_b, sem_b)
            pltpu.make_async_copy(
                mem_hbm.at[src_v.at[pl.ds(0, WIN)]], buf_a, sem_a).wait()
            pltpu.sync_copy(buf_a, out_hbm.at[pl.ds(wb, WIN), pl.ds(0, D_FEAT)])

            @pl.when(win + 1 < N_WIN)
            def _():
                pltpu.async_copy(
                    mem_hbm.at[src_v.at[pl.ds((win + 1) * WIN, WIN)]],
                    buf_a, sem_a)

            pltpu.make_async_copy(
                mem_hbm.at[dst_v.at[pl.ds(0, WIN)]], buf_b, sem_b).wait()
            pltpu.sync_copy(
                buf_b, out_hbm.at[pl.ds(wb, WIN), pl.ds(D_FEAT, D_FEAT)])

    return k(memory, src, dst)


def kernel(memory, last_update, events_features, time_w, time_b, timestamps,
           src_nodes, dst_nodes, event_indices, idx):
    dt = _sc_dt(last_update, timestamps, idx.astype(jnp.int32))
    ff = _sc_feats(events_features, event_indices.astype(jnp.int32))
    delta_t = _tc_delta_t(
        dt, time_w.reshape(TIME_DIM, 1), time_b.reshape(TIME_DIM, 1))
    out256 = _sc_assemble(
        memory, src_nodes.astype(jnp.int32), dst_nodes.astype(jnp.int32))
    return jnp.concatenate([out256, delta_t.T, ff], axis=1)
